# baseline (speedup 1.0000x reference)
import functools

import jax
import jax.numpy as jnp
from jax import lax
from jax.experimental import pallas as pl
from jax.experimental.pallas import tpu as pltpu
from jax.experimental.pallas import tpu_sc as plsc

N = 10000
E = 320000
D = 128

NC = 2
NS = 16
NW = NC * NS
G = 128
GPT = (-(-E // (NW * G)) + 7) // 8 * 8
E_PAD = NW * GPT * G
GPT0 = 32
GPT1 = 2 * GPT - GPT0
IC = 4
NCH0 = GPT0 // IC
NCH1 = GPT1 // IC
assert NCH0 % 2 == 0 and NCH1 % 2 == 0
N_ACC = 10112
RPT = N_ACC // NS
D1 = D + 16


def _make_segsum(Dw: int):
    mesh = plsc.VectorSubcoreMesh(core_axis_name="c", subcore_axis_name="s")

    @functools.partial(
        pl.kernel,
        out_type=jax.ShapeDtypeStruct((NC, N_ACC, Dw), jnp.float32),
        mesh=mesh,
        scratch_types=[
            pltpu.VMEM((2, IC, G), jnp.int32),
            pltpu.VMEM((2, IC, G), jnp.int32),
            pltpu.VMEM((G, Dw), jnp.float32),
            pltpu.VMEM((G, Dw), jnp.float32),
            pltpu.VMEM_SHARED((N_ACC, Dw), jnp.float32),
            pltpu.SemaphoreType.DMA,
            pltpu.SemaphoreType.DMA,
            pltpu.SemaphoreType.DMA,
            pltpu.SemaphoreType.DMA,
            pltpu.SemaphoreType.DMA,
            pltpu.SemaphoreType.DMA,
        ],
        compiler_params=pltpu.CompilerParams(use_tc_tiling_on_sc=False),
    )
    def segsum(table, src2d, dst2d, out, idx_s, idx_d, rows0, rows1, acc,
               semg0, semg1, sems0, sems1, semi0, semi1):
        c = lax.axis_index("c")
        s = lax.axis_index("s")
        rows = (rows0, rows1)
        semg = (semg0, semg1)
        sems = (sems0, sems1)
        semi = (semi0, semi1)

        gbase = jnp.where(c == 0, s * GPT0, NS * GPT0 + s * GPT1)
        ncht = jnp.where(c == 0, NCH0, NCH1)

        def src_sl(t):
            return src2d.at[pl.ds(gbase + t * IC, IC)]

        def dst_sl(t):
            return dst2d.at[pl.ds(gbase + t * IC, IC)]

        pltpu.async_copy(src_sl(0), idx_s.at[0], semi0)
        pltpu.async_copy(dst_sl(0), idx_d.at[0], semi0)

        def zero_row(i, carry):
            for j in range(Dw // 16):
                rows0[i, pl.ds(j * 16, 16)] = jnp.zeros((16,), jnp.float32)
            return carry

        lax.fori_loop(0, G, zero_row, 0)
        for k in range(RPT // G):
            pltpu.sync_copy(rows0, acc.at[pl.ds(s * RPT + k * G, G)])
        rem = RPT % G
        if rem:
            pltpu.sync_copy(rows0.at[pl.ds(0, rem)],
                            acc.at[pl.ds(s * RPT + (RPT // G) * G, rem)])
        plsc.subcore_barrier()

        def chunk(t, p):
            pltpu.make_async_copy(src_sl(t), idx_s.at[p], semi[p]).wait()
            pltpu.make_async_copy(dst_sl(t), idx_d.at[p], semi[p]).wait()
            for i in range(IC):
                b = i % 2

                def _wait_scatter(b=b, i=i):
                    pltpu.make_async_copy(rows[b], acc.at[idx_d.at[p].at[i]],
                                          sems[b]).wait()

                if i >= 2:
                    _wait_scatter()
                else:
                    pl.when(t > 0)(_wait_scatter)

                pltpu.async_copy(table.at[idx_s.at[p].at[i]], rows[b], semg[b])

                pp, pi = (p, i - 1) if i > 0 else (1 - p, IC - 1)

                def _scatter_prev(pp=pp, pi=pi, b=b):
                    pltpu.make_async_copy(table.at[idx_s.at[pp].at[pi]],
                                          rows[1 - b], semg[1 - b]).wait()
                    pltpu.async_copy(rows[1 - b], acc.at[idx_d.at[pp].at[pi]],
                                     sems[1 - b], add=True)

                if i == 0:
                    pl.when(t > 0)(_scatter_prev)
                else:
                    _scatter_prev()

                if i == 1:
                    def _prefetch():
                        pltpu.async_copy(src_sl(t + 1), idx_s.at[1 - p],
                                         semi[1 - p])
                        pltpu.async_copy(dst_sl(t + 1), idx_d.at[1 - p],
                                         semi[1 - p])

                    pl.when(t < ncht - 1)(_prefetch)

        def chunk_body(t, carry):
            pl.when(t % 2 == 0)(lambda: chunk(t, 0))
            pl.when(t % 2 == 1)(lambda: chunk(t, 1))
            return carry

        lax.fori_loop(0, ncht, chunk_body, 0)

        lp = 1
        lb = (IC - 1) % 2
        pltpu.make_async_copy(table.at[idx_s.at[lp].at[IC - 1]],
                              rows[lb], semg[lb]).wait()
        pltpu.sync_copy(rows[lb], acc.at[idx_d.at[lp].at[IC - 1]], add=True)
        pltpu.make_async_copy(rows[1 - lb], acc.at[idx_d.at[lp].at[IC - 2]],
                              sems[1 - lb]).wait()

        plsc.subcore_barrier()
        pltpu.sync_copy(acc.at[pl.ds(s * RPT, RPT)],
                        out.at[c].at[pl.ds(s * RPT, RPT)])

    return segsum


_segsum_l1 = _make_segsum(D1)
_segsum_l2 = _make_segsum(D)

BR = 1000


def _aug_mm_body(x_ref, w_ref, o_ref):
    y = lax.dot_general(x_ref[...], w_ref[...], (((1,), (1,)), ((), ())),
                        preferred_element_type=jnp.float32)
    o_ref[:, :D] = y
    col = lax.broadcasted_iota(jnp.int32, (BR, D1 - D), 1)
    o_ref[:, D:] = jnp.where(col == 0, 1.0, 0.0)


def _mid_body(p_ref, x_ref, w1r_ref, b1l_ref, w2l_ref, h_ref, y2_ref, inv_ref):
    sums = p_ref[0, :, :D] + p_ref[1, :, :D]
    cnt = p_ref[0, :, D:D + 1] + p_ref[1, :, D:D + 1]
    inv = 1.0 / jnp.maximum(cnt, 1.0)
    inv_ref[...] = inv
    xr = lax.dot_general(x_ref[...], w1r_ref[...], (((1,), (1,)), ((), ())),
                         preferred_element_type=jnp.float32)
    h = jnp.maximum(sums * inv + b1l_ref[...] + xr, 0.0)
    h_ref[...] = h
    y2_ref[...] = lax.dot_general(h, w2l_ref[...], (((1,), (1,)), ((), ())),
                                  preferred_element_type=jnp.float32)


def _out_body(p2_ref, inv_ref, h_ref, w2r_ref, b2l_ref, o_ref):
    sums = p2_ref[0] + p2_ref[1]
    inv = inv_ref[...]
    hr = lax.dot_general(h_ref[...], w2r_ref[...], (((1,), (1,)), ((), ())),
                         preferred_element_type=jnp.float32)
    o_ref[...] = sums * inv + b2l_ref[...] + hr


def _full(shape):
    return pl.BlockSpec(shape, lambda i: tuple(0 for _ in shape))


def kernel(x, edge_index, W1l, b1l, W1r, W2l, b2l, W2r):
    pad = E_PAD - E
    src2d = jnp.concatenate([edge_index[0], jnp.zeros((pad,), jnp.int32)]).reshape(-1, G)
    dst2d = jnp.concatenate([edge_index[1], jnp.full((pad,), N, jnp.int32)]).reshape(-1, G)
    b1l2 = b1l.reshape(1, D)
    b2l2 = b2l.reshape(1, D)
    grid = (N // BR,)

    y1 = pl.pallas_call(
        _aug_mm_body,
        grid=grid,
        in_specs=[pl.BlockSpec((BR, D), lambda i: (i, 0)), _full((D, D))],
        out_specs=pl.BlockSpec((BR, D1), lambda i: (i, 0)),
        out_shape=jax.ShapeDtypeStruct((N, D1), jnp.float32),
    )(x, W1l)

    p1 = _segsum_l1(y1, src2d, dst2d)

    h, y2, inv = pl.pallas_call(
        _mid_body,
        grid=grid,
        in_specs=[
            pl.BlockSpec((NC, BR, D1), lambda i: (0, i, 0)),
            pl.BlockSpec((BR, D), lambda i: (i, 0)),
            _full((D, D)),
            _full((1, D)),
            _full((D, D)),
        ],
        out_specs=[
            pl.BlockSpec((BR, D), lambda i: (i, 0)),
            pl.BlockSpec((BR, D), lambda i: (i, 0)),
            pl.BlockSpec((BR, 1), lambda i: (i, 0)),
        ],
        out_shape=[
            jax.ShapeDtypeStruct((N, D), jnp.float32),
            jax.ShapeDtypeStruct((N, D), jnp.float32),
            jax.ShapeDtypeStruct((N, 1), jnp.float32),
        ],
    )(p1, x, W1r, b1l2, W2l)

    p2 = _segsum_l2(y2, src2d, dst2d)

    out = pl.pallas_call(
        _out_body,
        grid=grid,
        in_specs=[
            pl.BlockSpec((NC, BR, D), lambda i: (0, i, 0)),
            pl.BlockSpec((BR, 1), lambda i: (i, 0)),
            pl.BlockSpec((BR, D), lambda i: (i, 0)),
            _full((D, D)),
            _full((1, D)),
        ],
        out_specs=pl.BlockSpec((BR, D), lambda i: (i, 0)),
        out_shape=jax.ShapeDtypeStruct((N, D), jnp.float32),
    )(p2, inv, h, W2r, b2l2)

    return out


# speedup vs baseline: 3.4712x; 3.4712x over previous
import functools

import jax
import jax.numpy as jnp
from jax import lax
from jax.experimental import pallas as pl
from jax.experimental.pallas import tpu as pltpu
from jax.experimental.pallas import tpu_sc as plsc

N = 10000
E = 320000
D = 128

NC = 2
NS = 16
NW = NC * NS
G = 128
GPT = (-(-E // (NW * G)) + 7) // 8 * 8
E_PAD = NW * GPT * G
GPT0 = GPT
GPT1 = 2 * GPT - GPT0
IC = 4
NCH0 = GPT0 // IC
NCH1 = GPT1 // IC
assert NCH0 % 2 == 0 and NCH1 % 2 == 0
N_ACC = 10112
RPT = N_ACC // NS
D1 = D + 16


def _make_segsum(Dw: int):
    mesh = plsc.VectorSubcoreMesh(core_axis_name="c", subcore_axis_name="s")

    @functools.partial(
        pl.kernel,
        out_type=jax.ShapeDtypeStruct((NC, N_ACC, Dw), jnp.float32),
        mesh=mesh,
        scratch_types=[
            pltpu.VMEM((2, IC, G), jnp.int32),
            pltpu.VMEM((2, IC, G), jnp.int32),
            pltpu.VMEM((G, Dw), jnp.float32),
            pltpu.VMEM((G, Dw), jnp.float32),
            pltpu.VMEM_SHARED((N_ACC, Dw), jnp.float32),
            pltpu.SemaphoreType.DMA,
            pltpu.SemaphoreType.DMA,
            pltpu.SemaphoreType.DMA,
            pltpu.SemaphoreType.DMA,
            pltpu.SemaphoreType.DMA,
            pltpu.SemaphoreType.DMA,
        ],
        compiler_params=pltpu.CompilerParams(use_tc_tiling_on_sc=False),
    )
    def segsum(table, src2d, dst2d, out, idx_s, idx_d, rows0, rows1, acc,
               semg0, semg1, sems0, sems1, semi0, semi1):
        c = lax.axis_index("c")
        s = lax.axis_index("s")
        rows = (rows0, rows1)
        semg = (semg0, semg1)
        sems = (sems0, sems1)
        semi = (semi0, semi1)

        gbase = jnp.where(c == 0, s * GPT0, NS * GPT0 + s * GPT1)
        ncht = jnp.where(c == 0, NCH0, NCH1)

        def src_sl(t):
            return src2d.at[pl.ds(gbase + t * IC, IC)]

        def dst_sl(t):
            return dst2d.at[pl.ds(gbase + t * IC, IC)]

        pltpu.async_copy(src_sl(0), idx_s.at[0], semi0)
        pltpu.async_copy(dst_sl(0), idx_d.at[0], semi0)

        def zero_row(i, carry):
            for j in range(Dw // 16):
                rows0[i, pl.ds(j * 16, 16)] = jnp.zeros((16,), jnp.float32)
            return carry

        lax.fori_loop(0, G, zero_row, 0)
        for k in range(RPT // G):
            pltpu.sync_copy(rows0, acc.at[pl.ds(s * RPT + k * G, G)])
        rem = RPT % G
        if rem:
            pltpu.sync_copy(rows0.at[pl.ds(0, rem)],
                            acc.at[pl.ds(s * RPT + (RPT // G) * G, rem)])
        plsc.subcore_barrier()

        def chunk(t, p):
            pltpu.make_async_copy(src_sl(t), idx_s.at[p], semi[p]).wait()
            pltpu.make_async_copy(dst_sl(t), idx_d.at[p], semi[p]).wait()
            for i in range(IC):
                b = i % 2

                def _wait_scatter(b=b, i=i):
                    pltpu.make_async_copy(rows[b], acc.at[idx_d.at[p].at[i]],
                                          sems[b]).wait()

                if i >= 2:
                    _wait_scatter()
                else:
                    pl.when(t > 0)(_wait_scatter)

                pltpu.async_copy(table.at[idx_s.at[p].at[i]], rows[b], semg[b])

                pp, pi = (p, i - 1) if i > 0 else (1 - p, IC - 1)

                def _scatter_prev(pp=pp, pi=pi, b=b):
                    pltpu.make_async_copy(table.at[idx_s.at[pp].at[pi]],
                                          rows[1 - b], semg[1 - b]).wait()
                    pltpu.async_copy(rows[1 - b], acc.at[idx_d.at[pp].at[pi]],
                                     sems[1 - b], add=True)

                if i == 0:
                    pl.when(t > 0)(_scatter_prev)
                else:
                    _scatter_prev()

                if i == 1:
                    def _prefetch():
                        pltpu.async_copy(src_sl(t + 1), idx_s.at[1 - p],
                                         semi[1 - p])
                        pltpu.async_copy(dst_sl(t + 1), idx_d.at[1 - p],
                                         semi[1 - p])

                    pl.when(t < ncht - 1)(_prefetch)

        def chunk_body(t, carry):
            pl.when(t % 2 == 0)(lambda: chunk(t, 0))
            pl.when(t % 2 == 1)(lambda: chunk(t, 1))
            return carry

        lax.fori_loop(0, ncht, chunk_body, 0)

        lp = 1
        lb = (IC - 1) % 2
        pltpu.make_async_copy(table.at[idx_s.at[lp].at[IC - 1]],
                              rows[lb], semg[lb]).wait()
        pltpu.sync_copy(rows[lb], acc.at[idx_d.at[lp].at[IC - 1]], add=True)
        pltpu.make_async_copy(rows[1 - lb], acc.at[idx_d.at[lp].at[IC - 2]],
                              sems[1 - lb]).wait()

        plsc.subcore_barrier()
        pltpu.sync_copy(acc.at[pl.ds(s * RPT, RPT)],
                        out.at[c].at[pl.ds(s * RPT, RPT)])

    return segsum


_segsum_l1 = _make_segsum(D1)
_segsum_l2 = _make_segsum(D)

BR = 1000


def _aug_mm_body(x_ref, w_ref, o_ref):
    y = lax.dot_general(x_ref[...], w_ref[...], (((1,), (1,)), ((), ())),
                        preferred_element_type=jnp.float32)
    o_ref[:, :D] = y
    col = lax.broadcasted_iota(jnp.int32, (BR, D1 - D), 1)
    o_ref[:, D:] = jnp.where(col == 0, 1.0, 0.0)


def _mid_body(p_ref, x_ref, w1r_ref, b1l_ref, w2l_ref, h_ref, y2_ref, inv_ref):
    sums = p_ref[0, :, :D] + p_ref[1, :, :D]
    cnt = p_ref[0, :, D:D + 1] + p_ref[1, :, D:D + 1]
    inv = 1.0 / jnp.maximum(cnt, 1.0)
    inv_ref[...] = inv
    xr = lax.dot_general(x_ref[...], w1r_ref[...], (((1,), (1,)), ((), ())),
                         preferred_element_type=jnp.float32)
    h = jnp.maximum(sums * inv + b1l_ref[...] + xr, 0.0)
    h_ref[...] = h
    y2_ref[...] = lax.dot_general(h, w2l_ref[...], (((1,), (1,)), ((), ())),
                                  preferred_element_type=jnp.float32)


def _out_body(p2_ref, inv_ref, h_ref, w2r_ref, b2l_ref, o_ref):
    sums = p2_ref[0] + p2_ref[1]
    inv = inv_ref[...]
    hr = lax.dot_general(h_ref[...], w2r_ref[...], (((1,), (1,)), ((), ())),
                         preferred_element_type=jnp.float32)
    o_ref[...] = sums * inv + b2l_ref[...] + hr


def _full(shape):
    return pl.BlockSpec(shape, lambda i: tuple(0 for _ in shape))


def kernel(x, edge_index, W1l, b1l, W1r, W2l, b2l, W2r):
    pad = E_PAD - E
    pad_src = jnp.arange(pad, dtype=jnp.int32) % N
    pad_dst = N + jnp.arange(pad, dtype=jnp.int32) % (N_ACC - N)
    src2d = jnp.concatenate([edge_index[0], pad_src]).reshape(-1, G)
    dst2d = jnp.concatenate([edge_index[1], pad_dst]).reshape(-1, G)
    b1l2 = b1l.reshape(1, D)
    b2l2 = b2l.reshape(1, D)
    grid = (N // BR,)

    y1 = pl.pallas_call(
        _aug_mm_body,
        grid=grid,
        in_specs=[pl.BlockSpec((BR, D), lambda i: (i, 0)), _full((D, D))],
        out_specs=pl.BlockSpec((BR, D1), lambda i: (i, 0)),
        out_shape=jax.ShapeDtypeStruct((N, D1), jnp.float32),
    )(x, W1l)

    p1 = _segsum_l1(y1, src2d, dst2d)

    h, y2, inv = pl.pallas_call(
        _mid_body,
        grid=grid,
        in_specs=[
            pl.BlockSpec((NC, BR, D1), lambda i: (0, i, 0)),
            pl.BlockSpec((BR, D), lambda i: (i, 0)),
            _full((D, D)),
            _full((1, D)),
            _full((D, D)),
        ],
        out_specs=[
            pl.BlockSpec((BR, D), lambda i: (i, 0)),
            pl.BlockSpec((BR, D), lambda i: (i, 0)),
            pl.BlockSpec((BR, 1), lambda i: (i, 0)),
        ],
        out_shape=[
            jax.ShapeDtypeStruct((N, D), jnp.float32),
            jax.ShapeDtypeStruct((N, D), jnp.float32),
            jax.ShapeDtypeStruct((N, 1), jnp.float32),
        ],
    )(p1, x, W1r, b1l2, W2l)

    p2 = _segsum_l2(y2, src2d, dst2d)

    out = pl.pallas_call(
        _out_body,
        grid=grid,
        in_specs=[
            pl.BlockSpec((NC, BR, D), lambda i: (0, i, 0)),
            pl.BlockSpec((BR, 1), lambda i: (i, 0)),
            pl.BlockSpec((BR, D), lambda i: (i, 0)),
            _full((D, D)),
            _full((1, D)),
        ],
        out_specs=pl.BlockSpec((BR, D), lambda i: (i, 0)),
        out_shape=jax.ShapeDtypeStruct((N, D), jnp.float32),
    )(p2, inv, h, W2r, b2l2)

    return out


# speedup vs baseline: 3.8430x; 1.1071x over previous
import functools

import jax
import jax.numpy as jnp
from jax import lax
from jax.experimental import pallas as pl
from jax.experimental.pallas import tpu as pltpu
from jax.experimental.pallas import tpu_sc as plsc

N = 10000
E = 320000
D = 128

NC = 2
NS = 16
NW = NC * NS
G = 128
GPT = (-(-E // (NW * G)) + 7) // 8 * 8
E_PAD = NW * GPT * G
IC = 4
NCH = GPT // IC
N_ACC = 10112
RPT = N_ACC // NS
CW = 16


def _make_segsum(with_counts: bool):
    mesh = plsc.VectorSubcoreMesh(core_axis_name="c", subcore_axis_name="s")

    out_type = [jax.ShapeDtypeStruct((NC, N_ACC, D), jnp.float32)]
    scratch = [
        pltpu.VMEM((2, IC, G), jnp.int32),
        pltpu.VMEM((2, IC, G), jnp.int32),
        pltpu.VMEM((G, D), jnp.float32),
        pltpu.VMEM((G, D), jnp.float32),
        pltpu.VMEM_SHARED((N_ACC, D), jnp.float32),
        pltpu.SemaphoreType.DMA,
        pltpu.SemaphoreType.DMA,
        pltpu.SemaphoreType.DMA,
        pltpu.SemaphoreType.DMA,
        pltpu.SemaphoreType.DMA,
        pltpu.SemaphoreType.DMA,
    ]
    if with_counts:
        out_type.append(jax.ShapeDtypeStruct((NC, N_ACC, CW), jnp.float32))
        scratch += [
            pltpu.VMEM((G, CW), jnp.float32),
            pltpu.VMEM_SHARED((N_ACC, CW), jnp.float32),
            pltpu.SemaphoreType.DMA,
        ]

    @functools.partial(pl.kernel,
                       out_type=tuple(out_type) if with_counts else out_type[0],
                       mesh=mesh,
                       scratch_types=scratch,
                       compiler_params=pltpu.CompilerParams(
                           use_tc_tiling_on_sc=False))
    def segsum(table, src2d, dst2d, *refs):
        if with_counts:
            (out, outc, idx_s, idx_d, rows0, rows1, acc,
             semg0, semg1, sems0, sems1, semi0, semi1,
             ones, accc, semc) = refs
        else:
            (out, idx_s, idx_d, rows0, rows1, acc,
             semg0, semg1, sems0, sems1, semi0, semi1) = refs
        c = lax.axis_index("c")
        s = lax.axis_index("s")
        rows = (rows0, rows1)
        semg = (semg0, semg1)
        sems = (sems0, sems1)
        semi = (semi0, semi1)
        gbase = (s * NC + c) * GPT

        def src_sl(t):
            return src2d.at[pl.ds(gbase + t * IC, IC)]

        def dst_sl(t):
            return dst2d.at[pl.ds(gbase + t * IC, IC)]

        pltpu.async_copy(src_sl(0), idx_s.at[0], semi0)
        pltpu.async_copy(dst_sl(0), idx_d.at[0], semi0)

        def zero_row(i, carry):
            for j in range(D // 16):
                rows0[i, pl.ds(j * 16, 16)] = jnp.zeros((16,), jnp.float32)
            return carry

        lax.fori_loop(0, G, zero_row, 0)
        nfull, rem = RPT // G, RPT % G
        for k in range(nfull):
            pltpu.sync_copy(rows0, acc.at[pl.ds(s * RPT + k * G, G)])
        if rem:
            pltpu.sync_copy(rows0.at[pl.ds(0, rem)],
                            acc.at[pl.ds(s * RPT + nfull * G, rem)])

        if with_counts:
            def zero_ones(i, carry):
                ones[i, pl.ds(0, CW)] = jnp.zeros((CW,), jnp.float32)
                return carry

            lax.fori_loop(0, G, zero_ones, 0)
            for k in range(nfull):
                pltpu.sync_copy(ones.at[pl.ds(0, G)],
                                accc.at[pl.ds(s * RPT + k * G, G)])
            if rem:
                pltpu.sync_copy(ones.at[pl.ds(0, rem)],
                                accc.at[pl.ds(s * RPT + nfull * G, rem)])

            def one_ones(i, carry):
                ones[i, pl.ds(0, CW)] = jnp.full((CW,), 1.0, jnp.float32)
                return carry

            lax.fori_loop(0, G, one_ones, 0)
        plsc.subcore_barrier()

        def chunk(t, p):
            pltpu.make_async_copy(src_sl(t), idx_s.at[p], semi[p]).wait()
            pltpu.make_async_copy(dst_sl(t), idx_d.at[p], semi[p]).wait()
            for i in range(IC):
                b = i % 2

                def _wait_scatter(b=b, i=i):
                    pltpu.make_async_copy(rows[b], acc.at[idx_d.at[p].at[i]],
                                          sems[b]).wait()

                if i >= 2:
                    _wait_scatter()
                else:
                    pl.when(t > 0)(_wait_scatter)

                if with_counts and i == 0:
                    def _drain_counts():
                        for j in range(IC):
                            pltpu.make_async_copy(
                                ones, accc.at[idx_d.at[1 - p].at[j]],
                                semc).wait()

                    pl.when(t > 0)(_drain_counts)

                pltpu.async_copy(table.at[idx_s.at[p].at[i]], rows[b], semg[b])

                pp, pi = (p, i - 1) if i > 0 else (1 - p, IC - 1)

                def _scatter_prev(pp=pp, pi=pi, b=b):
                    pltpu.make_async_copy(table.at[idx_s.at[pp].at[pi]],
                                          rows[1 - b], semg[1 - b]).wait()
                    pltpu.async_copy(rows[1 - b], acc.at[idx_d.at[pp].at[pi]],
                                     sems[1 - b], add=True)

                if i == 0:
                    pl.when(t > 0)(_scatter_prev)
                else:
                    _scatter_prev()

                if with_counts:
                    def _scatter_cnt(i=i):
                        pltpu.async_copy(ones, accc.at[idx_d.at[p].at[i]],
                                         semc, add=True)

                    _scatter_cnt()

                if i == 1:
                    def _prefetch():
                        pltpu.async_copy(src_sl(t + 1), idx_s.at[1 - p],
                                         semi[1 - p])
                        pltpu.async_copy(dst_sl(t + 1), idx_d.at[1 - p],
                                         semi[1 - p])

                    pl.when(t < NCH - 1)(_prefetch)

        def chunk_body(t, carry):
            pl.when(t % 2 == 0)(lambda: chunk(t, 0))
            pl.when(t % 2 == 1)(lambda: chunk(t, 1))
            return carry

        lax.fori_loop(0, NCH, chunk_body, 0)

        lp = 1
        lb = (IC - 1) % 2
        pltpu.make_async_copy(table.at[idx_s.at[lp].at[IC - 1]],
                              rows[lb], semg[lb]).wait()
        pltpu.sync_copy(rows[lb], acc.at[idx_d.at[lp].at[IC - 1]], add=True)
        pltpu.make_async_copy(rows[1 - lb], acc.at[idx_d.at[lp].at[IC - 2]],
                              sems[1 - lb]).wait()
        if with_counts:
            for j in range(IC):
                pltpu.make_async_copy(ones, accc.at[idx_d.at[lp].at[j]],
                                      semc).wait()

        plsc.subcore_barrier()
        pltpu.sync_copy(acc.at[pl.ds(s * RPT, RPT)],
                        out.at[c].at[pl.ds(s * RPT, RPT)])
        if with_counts:
            pltpu.sync_copy(accc.at[pl.ds(s * RPT, RPT)],
                            outc.at[c].at[pl.ds(s * RPT, RPT)])

    return segsum


_segsum_l1 = _make_segsum(True)
_segsum_l2 = _make_segsum(False)

BR = 1000


def _mm_body(x_ref, w_ref, o_ref):
    o_ref[...] = lax.dot_general(x_ref[...], w_ref[...],
                                 (((1,), (1,)), ((), ())),
                                 preferred_element_type=jnp.float32)


def _mid_body(p_ref, cnt_ref, x_ref, w1r_ref, b1l_ref, w2l_ref,
              h_ref, y2_ref, inv_ref):
    sums = p_ref[0] + p_ref[1]
    cnt = cnt_ref[0, :, 0:1] + cnt_ref[1, :, 0:1]
    inv = 1.0 / jnp.maximum(cnt, 1.0)
    inv_ref[...] = inv
    xr = lax.dot_general(x_ref[...], w1r_ref[...], (((1,), (1,)), ((), ())),
                         preferred_element_type=jnp.float32)
    h = jnp.maximum(sums * inv + b1l_ref[...] + xr, 0.0)
    h_ref[...] = h
    y2_ref[...] = lax.dot_general(h, w2l_ref[...], (((1,), (1,)), ((), ())),
                                  preferred_element_type=jnp.float32)


def _out_body(p2_ref, inv_ref, h_ref, w2r_ref, b2l_ref, o_ref):
    sums = p2_ref[0] + p2_ref[1]
    inv = inv_ref[...]
    hr = lax.dot_general(h_ref[...], w2r_ref[...], (((1,), (1,)), ((), ())),
                         preferred_element_type=jnp.float32)
    o_ref[...] = sums * inv + b2l_ref[...] + hr


def _full(shape):
    return pl.BlockSpec(shape, lambda i: tuple(0 for _ in shape))


def kernel(x, edge_index, W1l, b1l, W1r, W2l, b2l, W2r):
    pad = E_PAD - E
    pad_src = jnp.arange(pad, dtype=jnp.int32) % N
    pad_dst = N + jnp.arange(pad, dtype=jnp.int32) % (N_ACC - N)
    src2d = jnp.concatenate([edge_index[0], pad_src]).reshape(-1, G)
    dst2d = jnp.concatenate([edge_index[1], pad_dst]).reshape(-1, G)
    b1l2 = b1l.reshape(1, D)
    b2l2 = b2l.reshape(1, D)
    grid = (N // BR,)

    y1 = pl.pallas_call(
        _mm_body,
        grid=grid,
        in_specs=[pl.BlockSpec((BR, D), lambda i: (i, 0)), _full((D, D))],
        out_specs=pl.BlockSpec((BR, D), lambda i: (i, 0)),
        out_shape=jax.ShapeDtypeStruct((N, D), jnp.float32),
    )(x, W1l)

    p1, cnt = _segsum_l1(y1, src2d, dst2d)

    h, y2, inv = pl.pallas_call(
        _mid_body,
        grid=grid,
        in_specs=[
            pl.BlockSpec((NC, BR, D), lambda i: (0, i, 0)),
            pl.BlockSpec((NC, BR, CW), lambda i: (0, i, 0)),
            pl.BlockSpec((BR, D), lambda i: (i, 0)),
            _full((D, D)),
            _full((1, D)),
            _full((D, D)),
        ],
        out_specs=[
            pl.BlockSpec((BR, D), lambda i: (i, 0)),
            pl.BlockSpec((BR, D), lambda i: (i, 0)),
            pl.BlockSpec((BR, 1), lambda i: (i, 0)),
        ],
        out_shape=[
            jax.ShapeDtypeStruct((N, D), jnp.float32),
            jax.ShapeDtypeStruct((N, D), jnp.float32),
            jax.ShapeDtypeStruct((N, 1), jnp.float32),
        ],
    )(p1, cnt, x, W1r, b1l2, W2l)

    p2 = _segsum_l2(y2, src2d, dst2d)

    out = pl.pallas_call(
        _out_body,
        grid=grid,
        in_specs=[
            pl.BlockSpec((NC, BR, D), lambda i: (0, i, 0)),
            pl.BlockSpec((BR, 1), lambda i: (i, 0)),
            pl.BlockSpec((BR, D), lambda i: (i, 0)),
            _full((D, D)),
            _full((1, D)),
        ],
        out_specs=pl.BlockSpec((BR, D), lambda i: (i, 0)),
        out_shape=jax.ShapeDtypeStruct((N, D), jnp.float32),
    )(p2, inv, h, W2r, b2l2)

    return out
